# Initial kernel scaffold; baseline (speedup 1.0000x reference)
import jax
import jax.numpy as jnp
from jax.experimental import pallas as pl


def kernel(x, scale_weights, centers, widths):
    raise NotImplementedError("write your pallas kernel here")


# speedup vs baseline: 11.4767x; 11.4767x over previous
import jax
import jax.numpy as jnp
from jax.experimental import pallas as pl
from jax.experimental.pallas import tpu as pltpu

_EPS = 1e-6
_MAXR = 4


def _body(sw_ref, x_ref, cen_ref, wid_ref, o_ref):
    x = x_ref[0]
    H, W2, L = x.shape
    half = L // 2
    xa = jnp.abs(x) + _EPS

    zc = jnp.zeros((H, 2, L), jnp.float32)
    ap = jnp.concatenate([zc, xa, zc], axis=1)

    def s(t):
        return ap[:, 2 + t:2 + t + W2, :]

    def s0(t):
        return ap[:, 2 + t:2 + t + W2, :half]

    def s1(t):
        return ap[:, 2 + t:2 + t + W2, half:]

    p1 = jnp.concatenate([s1(-1) + s1(0), s0(0) + s0(1)], axis=-1)
    p3 = jnp.concatenate([s1(-2) + s1(1), s0(-1) + s0(2)], axis=-1)

    h1 = xa + p1
    h2 = h1 + s(-1) + s(1)
    h3 = h2 + p3
    h4 = h3 + s(-2) + s(2)

    alpha = jnp.zeros((H, W2, L), jnp.float32)
    for r, h in ((1, h1), (2, h2), (3, h3), (4, h4)):
        zr = jnp.zeros((r, W2, L), jnp.float32)
        hp = jnp.concatenate([zr, h, zr], axis=0)
        mu = h
        for d in range(1, r + 1):
            mu = mu + hp[r - d:r - d + H] + hp[r + d:r + d + H]
        alpha = alpha + sw_ref[r - 1] * jnp.log(mu)

    K = cen_ref.shape[0]
    acc = jnp.zeros((H, W2, L), jnp.float32)
    for k in range(K):
        ck = cen_ref[k, :].reshape(1, 1, L)
        wk = wid_ref[k, :].reshape(1, 1, L)
        dk = alpha - ck
        acc = acc + jnp.maximum(1.0 - wk * (dk * dk), 0.0)

    o_ref[0] = x + 1.0 / (1.0 + jnp.exp(-acc))


def kernel(x, scale_weights, centers, widths):
    B, H, W, C = x.shape
    K = centers.shape[1]
    W2, L = W // 2, 2 * C
    xr = x.reshape(B, H, W2, L)
    cen2 = jnp.tile(centers.T, (1, 2))
    wid2 = jnp.tile(widths.T, (1, 2))

    out = pl.pallas_call(
        _body,
        grid=(B,),
        in_specs=[
            pl.BlockSpec(memory_space=pltpu.SMEM),
            pl.BlockSpec((1, H, W2, L), lambda b: (b, 0, 0, 0)),
            pl.BlockSpec((K, L), lambda b: (0, 0)),
            pl.BlockSpec((K, L), lambda b: (0, 0)),
        ],
        out_specs=pl.BlockSpec((1, H, W2, L), lambda b: (b, 0, 0, 0)),
        out_shape=jax.ShapeDtypeStruct((B, H, W2, L), jnp.float32),
        compiler_params=pltpu.CompilerParams(
            dimension_semantics=("parallel",),
            vmem_limit_bytes=52 * 1024 * 1024,
        ),
        name="singularity_hist_recal",
    )(scale_weights, xr, cen2, wid2)
    return out.reshape(B, H, W, C)
